# baseline (device time: 175147 ns/iter reference)
import jax
import jax.numpy as jnp
from jax import lax
from jax.experimental import pallas as pl
from jax.experimental.pallas import tpu as pltpu

N_DEV = 16
N_TOK = 2048
D_MODEL = 512
D_HID = 1024
N_EXP = 64
E_LOCAL = N_EXP // N_DEV
CHUNK = N_TOK // N_DEV
N_HOP = N_DEV - 1

RINGS = (
    {"dstep": -1, "c0": 0, "w": 256},
    {"dstep": -1, "c0": 256, "w": 256},
    {"dstep": 1, "c0": 512, "w": 256},
    {"dstep": 1, "c0": 768, "w": 256},
)


def kernel(x, router_W, route_idx, expert_W):
    def body(x_ref, rw_ref, idx_ref, ew_ref, out_ref, acc_ref, *sc):
        my = lax.axis_index("i")

        def mod(v):
            return lax.rem(v + 4 * N_DEV, N_DEV)

        xv = x_ref[...]
        scores = jnp.dot(xv, rw_ref[...], preferred_element_type=jnp.float32)
        m = jnp.max(scores, axis=-1, keepdims=True)
        p = jnp.exp(scores - m)
        p = p / jnp.sum(p, axis=-1, keepdims=True)
        idx = idx_ref[...]
        e0 = idx[:, 0:1]
        e1 = idx[:, 1:2]
        cols = lax.broadcasted_iota(jnp.int32, (N_TOK, N_EXP), 1)
        g0 = jnp.sum(jnp.where(cols == e0, p, 0.0), axis=-1, keepdims=True)
        g1 = jnp.sum(jnp.where(cols == e1, p, 0.0), axis=-1, keepdims=True)
        gs = g0 + g1
        acc = jnp.zeros((N_TOK, D_HID), jnp.float32)
        for le in range(E_LOCAL):
            ge = my * E_LOCAL + le
            w = jnp.where(e0 == ge, g0 / gs, 0.0) + jnp.where(e1 == ge, g1 / gs, 0.0)
            acc = acc + jnp.dot(
                xv * w, ew_ref[le], preferred_element_type=jnp.float32
            )
        acc_ref[...] = acc

        def sc_of(ri):
            return sc[ri * 6 : (ri + 1) * 6]

        tgts = [mod(my - rg["dstep"]) for rg in RINGS]
        all_descs = []

        def start_rs(ri, s):
            rg = RINGS[ri]
            comm1, _, s1s, s1r, _, _ = sc_of(ri)
            c = mod(my + rg["dstep"] * s)
            d = pltpu.make_async_remote_copy(
                src_ref=acc_ref.at[
                    pl.ds(c * CHUNK, CHUNK), pl.ds(rg["c0"], rg["w"])
                ],
                dst_ref=comm1.at[s],
                send_sem=s1s.at[s],
                recv_sem=s1r.at[s],
                device_id=(tgts[ri],),
                device_id_type=pl.DeviceIdType.MESH,
            )
            d.start()
            all_descs.append(d)
            return d

        rs_descs = [[None] * N_HOP for _ in RINGS]
        for ri in range(len(RINGS)):
            rs_descs[ri][0] = start_rs(ri, 0)
        for s in range(N_HOP):
            for ri, rg in enumerate(RINGS):
                comm1, _, _, _, _, _ = sc_of(ri)
                rs_descs[ri][s].wait_recv()
                c = mod(my + rg["dstep"] * (s + 1))
                rows = pl.ds(c * CHUNK, CHUNK)
                cols_sl = pl.ds(rg["c0"], rg["w"])
                acc_ref[rows, cols_sl] = acc_ref[rows, cols_sl] + comm1[s]
                if s + 1 < N_HOP:
                    rs_descs[ri][s + 1] = start_rs(ri, s + 1)

        owns = [mod(my - rg["dstep"]) for rg in RINGS]
        for ri, rg in enumerate(RINGS):
            rows = pl.ds(owns[ri] * CHUNK, CHUNK)
            cols_sl = pl.ds(rg["c0"], rg["w"])
            out_ref[rows, cols_sl] = acc_ref[rows, cols_sl]

        def start_ag(ri, h):
            rg = RINGS[ri]
            _, comm2, _, _, s2s, s2r = sc_of(ri)
            if h == 0:
                src = acc_ref.at[
                    pl.ds(owns[ri] * CHUNK, CHUNK), pl.ds(rg["c0"], rg["w"])
                ]
            else:
                src = comm2.at[h - 1]
            d = pltpu.make_async_remote_copy(
                src_ref=src,
                dst_ref=comm2.at[h],
                send_sem=s2s.at[h],
                recv_sem=s2r.at[h],
                device_id=(tgts[ri],),
                device_id_type=pl.DeviceIdType.MESH,
            )
            d.start()
            all_descs.append(d)
            return d

        ag_descs = [[None] * N_HOP for _ in RINGS]
        for ri in range(len(RINGS)):
            ag_descs[ri][0] = start_ag(ri, 0)
        for h in range(N_HOP):
            for ri, rg in enumerate(RINGS):
                _, comm2, _, _, _, _ = sc_of(ri)
                ag_descs[ri][h].wait_recv()
                if h + 1 < N_HOP:
                    ag_descs[ri][h + 1] = start_ag(ri, h + 1)
                o = mod(owns[ri] + rg["dstep"] * (h + 1))
                out_ref[pl.ds(o * CHUNK, CHUNK), pl.ds(rg["c0"], rg["w"])] = (
                    comm2[h]
                )

        for d in all_descs:
            d.wait_send()

    scratch = [pltpu.VMEM((N_TOK, D_HID), jnp.float32)]
    for rg in RINGS:
        scratch += [
            pltpu.VMEM((N_HOP, CHUNK, rg["w"]), jnp.float32),
            pltpu.VMEM((N_HOP, CHUNK, rg["w"]), jnp.float32),
            pltpu.SemaphoreType.DMA((N_HOP,)),
            pltpu.SemaphoreType.DMA((N_HOP,)),
            pltpu.SemaphoreType.DMA((N_HOP,)),
            pltpu.SemaphoreType.DMA((N_HOP,)),
        ]

    return pl.pallas_call(
        body,
        out_shape=jax.ShapeDtypeStruct((N_TOK, D_HID), jnp.float32),
        in_specs=[pl.BlockSpec(memory_space=pltpu.VMEM)] * 4,
        out_specs=pl.BlockSpec(memory_space=pltpu.VMEM),
        scratch_shapes=scratch,
        compiler_params=pltpu.CompilerParams(
            vmem_limit_bytes=100 * 1024 * 1024,
        ),
    )(x, router_W, route_idx, expert_W)


# device time: 160982 ns/iter; 1.0880x vs baseline; 1.0880x over previous
import jax
import jax.numpy as jnp
from jax import lax
from jax.experimental import pallas as pl
from jax.experimental.pallas import tpu as pltpu

N_DEV = 16
N_TOK = 2048
D_MODEL = 512
D_HID = 1024
N_EXP = 64
E_LOCAL = N_EXP // N_DEV
CHUNK = N_TOK // N_DEV
N_HOP = N_DEV - 1

RINGS = (
    {"dstep": -1, "c0": 0, "w": 128},
    {"dstep": 1, "c0": 512, "w": 128},
    {"dstep": -1, "c0": 128, "w": 128},
    {"dstep": 1, "c0": 640, "w": 128},
    {"dstep": -1, "c0": 256, "w": 128},
    {"dstep": 1, "c0": 768, "w": 128},
    {"dstep": -1, "c0": 384, "w": 128},
    {"dstep": 1, "c0": 896, "w": 128},
)


def kernel(x, router_W, route_idx, expert_W):
    def body(x_ref, rw_ref, idx_ref, ew_ref, out_ref, acc_ref, *sc):
        my = lax.axis_index("i")

        def mod(v):
            return lax.rem(v + 4 * N_DEV, N_DEV)

        xv = x_ref[...]
        scores = jnp.dot(xv, rw_ref[...], preferred_element_type=jnp.float32)
        m = jnp.max(scores, axis=-1, keepdims=True)
        p = jnp.exp(scores - m)
        p = p / jnp.sum(p, axis=-1, keepdims=True)
        idx = idx_ref[...]
        e0 = idx[:, 0:1]
        e1 = idx[:, 1:2]
        cols = lax.broadcasted_iota(jnp.int32, (N_TOK, N_EXP), 1)
        g0 = jnp.sum(jnp.where(cols == e0, p, 0.0), axis=-1, keepdims=True)
        g1 = jnp.sum(jnp.where(cols == e1, p, 0.0), axis=-1, keepdims=True)
        gs = g0 + g1
        acc = jnp.zeros((N_TOK, D_HID), jnp.float32)
        for le in range(E_LOCAL):
            ge = my * E_LOCAL + le
            w = jnp.where(e0 == ge, g0 / gs, 0.0) + jnp.where(e1 == ge, g1 / gs, 0.0)
            acc = acc + jnp.dot(
                xv * w, ew_ref[le], preferred_element_type=jnp.float32
            )
        acc_ref[...] = acc

        def sc_of(ri):
            return sc[ri * 6 : (ri + 1) * 6]

        tgts = [mod(my - rg["dstep"]) for rg in RINGS]
        all_descs = []

        def start_rs(ri, s):
            rg = RINGS[ri]
            comm1, _, s1s, s1r, _, _ = sc_of(ri)
            c = mod(my + rg["dstep"] * s)
            d = pltpu.make_async_remote_copy(
                src_ref=acc_ref.at[
                    pl.ds(c * CHUNK, CHUNK), pl.ds(rg["c0"], rg["w"])
                ],
                dst_ref=comm1.at[s],
                send_sem=s1s.at[s],
                recv_sem=s1r.at[s],
                device_id=(tgts[ri],),
                device_id_type=pl.DeviceIdType.MESH,
            )
            d.start()
            all_descs.append(d)
            return d

        rs_descs = [[None] * N_HOP for _ in RINGS]
        for ri in range(len(RINGS)):
            rs_descs[ri][0] = start_rs(ri, 0)
        for s in range(N_HOP):
            for ri, rg in enumerate(RINGS):
                comm1, _, _, _, _, _ = sc_of(ri)
                rs_descs[ri][s].wait_recv()
                c = mod(my + rg["dstep"] * (s + 1))
                rows = pl.ds(c * CHUNK, CHUNK)
                cols_sl = pl.ds(rg["c0"], rg["w"])
                acc_ref[rows, cols_sl] = acc_ref[rows, cols_sl] + comm1[s]
                if s + 1 < N_HOP:
                    rs_descs[ri][s + 1] = start_rs(ri, s + 1)

        owns = [mod(my - rg["dstep"]) for rg in RINGS]
        for ri, rg in enumerate(RINGS):
            rows = pl.ds(owns[ri] * CHUNK, CHUNK)
            cols_sl = pl.ds(rg["c0"], rg["w"])
            out_ref[rows, cols_sl] = acc_ref[rows, cols_sl]

        def start_ag(ri, h):
            rg = RINGS[ri]
            _, comm2, _, _, s2s, s2r = sc_of(ri)
            if h == 0:
                src = acc_ref.at[
                    pl.ds(owns[ri] * CHUNK, CHUNK), pl.ds(rg["c0"], rg["w"])
                ]
            else:
                src = comm2.at[h - 1]
            d = pltpu.make_async_remote_copy(
                src_ref=src,
                dst_ref=comm2.at[h],
                send_sem=s2s.at[h],
                recv_sem=s2r.at[h],
                device_id=(tgts[ri],),
                device_id_type=pl.DeviceIdType.MESH,
            )
            d.start()
            all_descs.append(d)
            return d

        ag_descs = [[None] * N_HOP for _ in RINGS]
        for ri in range(len(RINGS)):
            ag_descs[ri][0] = start_ag(ri, 0)
        for h in range(N_HOP):
            for ri, rg in enumerate(RINGS):
                _, comm2, _, _, _, _ = sc_of(ri)
                ag_descs[ri][h].wait_recv()
                if h + 1 < N_HOP:
                    ag_descs[ri][h + 1] = start_ag(ri, h + 1)
                o = mod(owns[ri] + rg["dstep"] * (h + 1))
                out_ref[pl.ds(o * CHUNK, CHUNK), pl.ds(rg["c0"], rg["w"])] = (
                    comm2[h]
                )

        for d in all_descs:
            d.wait_send()

    scratch = [pltpu.VMEM((N_TOK, D_HID), jnp.float32)]
    for rg in RINGS:
        scratch += [
            pltpu.VMEM((N_HOP, CHUNK, rg["w"]), jnp.float32),
            pltpu.VMEM((N_HOP, CHUNK, rg["w"]), jnp.float32),
            pltpu.SemaphoreType.DMA((N_HOP,)),
            pltpu.SemaphoreType.DMA((N_HOP,)),
            pltpu.SemaphoreType.DMA((N_HOP,)),
            pltpu.SemaphoreType.DMA((N_HOP,)),
        ]

    return pl.pallas_call(
        body,
        out_shape=jax.ShapeDtypeStruct((N_TOK, D_HID), jnp.float32),
        in_specs=[pl.BlockSpec(memory_space=pltpu.VMEM)] * 4,
        out_specs=pl.BlockSpec(memory_space=pltpu.VMEM),
        scratch_shapes=scratch,
        compiler_params=pltpu.CompilerParams(
            vmem_limit_bytes=100 * 1024 * 1024,
        ),
    )(x, router_W, route_idx, expert_W)
